# baseline (speedup 1.0000x reference)
import functools

import jax
import jax.numpy as jnp
from jax import lax
from jax.experimental import pallas as pl
from jax.experimental.pallas import tpu as pltpu
from jax.experimental.pallas import tpu_sc as plsc

_F32 = jnp.float32
_HIGHEST = lax.Precision.HIGHEST


def _dot(a, b):
    return lax.dot_general(a, b, (((1,), (0,)), ((), ())),
                           precision=lax.Precision.DEFAULT,
                           preferred_element_type=_F32)


def _conv_block_body(xs_ref, w_ref, b_ref, feat_ref):
    acc = _dot(xs_ref[...], w_ref[...])
    feat_ref[...] = jnp.maximum(acc + b_ref[...], 0.0)


def _stage_a(xcat, w_stack, b1_row, *, interpret=False):
    m_rows, k3d = xcat.shape
    hid = w_stack.shape[1]
    bt = 80
    return pl.pallas_call(
        _conv_block_body,
        grid=(m_rows // bt,),
        in_specs=[
            pl.BlockSpec((bt, k3d), lambda i: (i, 0)),
            pl.BlockSpec((k3d, hid), lambda i: (0, 0)),
            pl.BlockSpec((1, hid), lambda i: (0, 0)),
        ],
        out_specs=pl.BlockSpec((bt, hid), lambda i: (i, 0)),
        out_shape=jax.ShapeDtypeStruct((m_rows, hid), _F32),
        interpret=interpret,
    )(xcat, w_stack, b1_row)


def _cas_block_body(f_ref, wc_ref, cas_ref, sm_ref):
    c = _dot(f_ref[...], wc_ref[...])
    cas_ref[...] = c
    m = jnp.max(c, axis=1, keepdims=True)
    e = jnp.exp(c - m)
    sm_ref[...] = e / jnp.sum(e, axis=1, keepdims=True)


def _stage_a2(feats, w_cas, *, interpret=False):
    m_rows, hid = feats.shape
    c = w_cas.shape[1]
    bt = 600
    return pl.pallas_call(
        _cas_block_body,
        grid=(m_rows // bt,),
        in_specs=[
            pl.BlockSpec((bt, hid), lambda i: (i, 0)),
            pl.BlockSpec((hid, c), lambda i: (0, 0)),
        ],
        out_specs=[
            pl.BlockSpec((bt, c), lambda i: (i, 0)),
            pl.BlockSpec((bt, c), lambda i: (i, 0)),
        ],
        out_shape=[
            jax.ShapeDtypeStruct((m_rows, c), _F32),
            jax.ShapeDtypeStruct((m_rows, c), _F32),
        ],
        interpret=interpret,
    )(feats, w_cas)


def _select_body(k_act, k_bkg, t, mag_ref, casbt_ref, casct_ref,
                 gidxa_ref, gidxb_ref, sa_ref, sb_ref, vals_scr):
    b = mag_ref.shape[0]
    bc = casct_ref.shape[0]
    ka_pad = gidxa_ref.shape[1]
    kb_pad = gidxb_ref.shape[1]
    neg_inf = _F32(-jnp.inf)
    iota_t = lax.broadcasted_iota(jnp.int32, (b, t), 1)
    iota_ka = lax.broadcasted_iota(jnp.int32, (b, ka_pad), 1)
    iota_kb = lax.broadcasted_iota(jnp.int32, (b, kb_pad), 1)
    row_base = lax.broadcasted_iota(jnp.int32, (b, 1), 0) * t

    mag = mag_ref[...]

    def act_step(k, carry):
        vals, acc = carry
        m = jnp.max(vals, axis=1, keepdims=True)
        idx = jnp.min(jnp.where(vals == m, iota_t, t), axis=1, keepdims=True)
        acc = acc + jnp.where(iota_ka == k, row_base + idx, 0)
        return jnp.where(iota_t == idx, neg_inf, vals), acc

    _, gidxa = lax.fori_loop(0, k_act, act_step,
                             (mag, jnp.zeros((b, ka_pad), jnp.int32)))
    gidxa_ref[...] = gidxa

    rev = jnp.max(mag, axis=1, keepdims=True) - mag

    def bkg_step(k, carry):
        vals, acc, msk = carry
        m = jnp.max(vals, axis=1, keepdims=True)
        idx = jnp.min(jnp.where(vals == m, iota_t, t), axis=1, keepdims=True)
        acc = acc + jnp.where(iota_kb == k, row_base + idx, 0)
        hit = (iota_t == idx)
        return jnp.where(hit, neg_inf, vals), acc, msk + hit.astype(_F32)

    _, gidxb, bkg_mask = lax.fori_loop(
        0, k_bkg, bkg_step,
        (rev, jnp.zeros((b, kb_pad), jnp.int32), jnp.zeros((b, t), _F32)))
    gidxb_ref[...] = gidxb

    sb = lax.dot_general(bkg_mask, casbt_ref[...], (((1,), (1,)), ((0,), (0,))),
                         precision=_HIGHEST,
                         preferred_element_type=_F32) / _F32(k_bkg)

    iota_ct = lax.broadcasted_iota(jnp.int32, (bc, t), 1)
    vals_scr[...] = casct_ref[...]

    def cas_step(k, acc):
        v = vals_scr[...]
        m = jnp.max(v, axis=1, keepdims=True)
        idx = jnp.min(jnp.where(v == m, iota_ct, t), axis=1, keepdims=True)
        vals_scr[...] = jnp.where(iota_ct == idx, neg_inf, v)
        return acc + m

    acc = lax.fori_loop(0, k_act, cas_step, jnp.zeros((bc, 1), _F32))
    sa = acc.reshape(sa_ref.shape) / _F32(k_act)

    def _softmax(v):
        e = jnp.exp(v - jnp.max(v, axis=1, keepdims=True))
        return e / jnp.sum(e, axis=1, keepdims=True)

    sa_ref[...] = _softmax(sa)
    sb_ref[...] = _softmax(sb)


def _stage_b(mag, cas_bt, cas_ct, k_act, k_bkg, ka_pad, kb_pad, *, interpret=False):
    b, t = mag.shape
    c = cas_bt.shape[2]
    bc = cas_ct.shape[0]
    return pl.pallas_call(
        functools.partial(_select_body, k_act, k_bkg, t),
        grid=(1,),
        in_specs=[
            pl.BlockSpec((b, t), lambda i: (0, 0)),
            pl.BlockSpec((b, t, c), lambda i: (0, 0, 0)),
            pl.BlockSpec((bc, t), lambda i: (0, 0)),
        ],
        out_specs=[
            pl.BlockSpec((b, ka_pad), lambda i: (0, 0)),
            pl.BlockSpec((b, kb_pad), lambda i: (0, 0)),
            pl.BlockSpec((b, c), lambda i: (0, 0)),
            pl.BlockSpec((b, c), lambda i: (0, 0)),
        ],
        out_shape=[
            jax.ShapeDtypeStruct((b, ka_pad), jnp.int32),
            jax.ShapeDtypeStruct((b, kb_pad), jnp.int32),
            jax.ShapeDtypeStruct((b, c), _F32),
            jax.ShapeDtypeStruct((b, c), _F32),
        ],
        scratch_shapes=[pltpu.VMEM((bc, t), _F32)],
        interpret=interpret,
    )(mag, cas_bt, cas_ct)


_SC_WORKERS = 32


def _sc_gather(table, idx_flat, chunk):
    n_rows, hid = idx_flat.shape[0], table.shape[1]
    rows_per = n_rows // _SC_WORKERS
    nchunks = rows_per // chunk
    mesh = plsc.VectorSubcoreMesh(core_axis_name="c", subcore_axis_name="s")

    @functools.partial(
        pl.kernel,
        out_type=jax.ShapeDtypeStruct((n_rows, hid), _F32),
        mesh=mesh,
        scratch_types=[
            pltpu.VMEM((chunk,), jnp.int32),
            pltpu.VMEM((chunk, hid), _F32),
            pltpu.SemaphoreType.DMA,
        ],
    )
    def gk(table_hbm, idx_hbm, out_hbm, idx_v, rows_v, sem):
        wid = lax.axis_index("s") * 2 + lax.axis_index("c")
        for h in range(nchunks):
            base = wid * rows_per + h * chunk
            pltpu.sync_copy(idx_hbm.at[pl.ds(base, chunk)], idx_v)
            pltpu.async_copy(table_hbm.at[idx_v], rows_v, sem).wait()
            pltpu.sync_copy(rows_v, out_hbm.at[pl.ds(base, chunk)])

    return gk(table, idx_flat)


def kernel(x, W1, b1, W2):
    b, t, d = x.shape
    hid = W1.shape[0]
    c = W2.shape[0]
    k_act = t // 8
    k_bkg = t // 6
    ka_pad = 96
    kb_pad = 128

    w_stack = jnp.transpose(W1, (2, 1, 0)).reshape(3 * d, hid)
    w_cas = jnp.transpose(W2[:, :, 0], (1, 0))
    b1_row = b1.reshape(1, hid)
    x_prev = jnp.pad(x, ((0, 0), (1, 0), (0, 0)))[:, :t].reshape(b * t, d)
    x_next = jnp.pad(x, ((0, 0), (0, 1), (0, 0)))[:, 1:].reshape(b * t, d)
    xcat = jnp.concatenate([x_prev, x.reshape(b * t, d), x_next], axis=1)

    feats = _stage_a(xcat, w_stack, b1_row)
    cas, cas_sm = _stage_a2(feats, w_cas)

    features = feats.reshape(b, t, hid)
    mag = jnp.linalg.norm(features, axis=2)

    cas_bt = cas.reshape(b, t, c)
    cas_ct = jnp.transpose(cas_bt, (0, 2, 1)).reshape(b * c, t)
    gidxa, gidxb, score_act, score_bkg = _stage_b(
        mag, cas_bt, cas_ct, k_act, k_bkg, ka_pad, kb_pad)

    act_rows = _sc_gather(feats, gidxa.reshape(b * ka_pad), 48)
    bkg_rows = _sc_gather(feats, gidxb.reshape(b * kb_pad), 32)

    feat_act = act_rows.reshape(b, ka_pad, hid)[:, :k_act]
    feat_bkg = bkg_rows.reshape(b, kb_pad, hid)[:, :k_bkg]
    cas_softmax = cas_sm.reshape(b, t, c)
    return (score_act, score_bkg, feat_act, feat_bkg, features, cas_softmax)


# speedup vs baseline: 1.2131x; 1.2131x over previous
import functools

import jax
import jax.numpy as jnp
from jax import lax
from jax.experimental import pallas as pl
from jax.experimental.pallas import tpu as pltpu
from jax.experimental.pallas import tpu_sc as plsc

_F32 = jnp.float32
_HIGHEST = lax.Precision.HIGHEST


def _dot(a, b):
    return lax.dot_general(a, b, (((1,), (0,)), ((), ())),
                           precision=lax.Precision.DEFAULT,
                           preferred_element_type=_F32)


def _conv_block_body(xp_ref, xc_ref, xn_ref, w_ref, b_ref, feat_ref, xs_scr):
    d = xp_ref.shape[1]
    xs_scr[:, 0:d] = xp_ref[...]
    xs_scr[:, d:2 * d] = xc_ref[...]
    xs_scr[:, 2 * d:3 * d] = xn_ref[...]
    acc = _dot(xs_scr[...], w_ref[...])
    feat_ref[...] = jnp.maximum(acc + b_ref[...], 0.0)


def _stage_a(x_prev, x_cur, x_next, w_stack, b1_row, *, interpret=False):
    m_rows, d = x_cur.shape
    hid = w_stack.shape[1]
    bt = 96
    x_spec = pl.BlockSpec((bt, d), lambda i: (i, 0))
    return pl.pallas_call(
        _conv_block_body,
        grid=(m_rows // bt,),
        in_specs=[
            x_spec, x_spec, x_spec,
            pl.BlockSpec((3 * d, hid), lambda i: (0, 0)),
            pl.BlockSpec((1, hid), lambda i: (0, 0)),
        ],
        out_specs=pl.BlockSpec((bt, hid), lambda i: (i, 0)),
        out_shape=jax.ShapeDtypeStruct((m_rows, hid), _F32),
        scratch_shapes=[pltpu.VMEM((bt, 3 * d), _F32)],
        interpret=interpret,
    )(x_prev, x_cur, x_next, w_stack, b1_row)


def _cas_block_body(f_ref, wc_ref, cas_ref, sm_ref):
    c = _dot(f_ref[...], wc_ref[...])
    cas_ref[...] = c
    m = jnp.max(c, axis=1, keepdims=True)
    e = jnp.exp(c - m)
    sm_ref[...] = e / jnp.sum(e, axis=1, keepdims=True)


def _stage_a2(feats, w_cas, *, interpret=False):
    m_rows, hid = feats.shape
    c = w_cas.shape[1]
    bt = 600
    return pl.pallas_call(
        _cas_block_body,
        grid=(m_rows // bt,),
        in_specs=[
            pl.BlockSpec((bt, hid), lambda i: (i, 0)),
            pl.BlockSpec((hid, c), lambda i: (0, 0)),
        ],
        out_specs=[
            pl.BlockSpec((bt, c), lambda i: (i, 0)),
            pl.BlockSpec((bt, c), lambda i: (i, 0)),
        ],
        out_shape=[
            jax.ShapeDtypeStruct((m_rows, c), _F32),
            jax.ShapeDtypeStruct((m_rows, c), _F32),
        ],
        interpret=interpret,
    )(feats, w_cas)


def _select_body(k_act, k_bkg, t, mag_ref, casbt_ref, casct_ref,
                 gidxa_ref, gidxb_ref, sa_ref, sb_ref, vals_scr):
    b = mag_ref.shape[0]
    bc = casct_ref.shape[0]
    ka_pad = gidxa_ref.shape[1]
    kb_pad = gidxb_ref.shape[1]
    neg_inf = _F32(-jnp.inf)
    iota_t = lax.broadcasted_iota(jnp.int32, (b, t), 1)
    iota_ka = lax.broadcasted_iota(jnp.int32, (b, ka_pad), 1)
    iota_kb = lax.broadcasted_iota(jnp.int32, (b, kb_pad), 1)
    row_base = lax.broadcasted_iota(jnp.int32, (b, 1), 0) * t

    mag = mag_ref[...]

    def act_step(k, carry):
        vals, acc = carry
        m = jnp.max(vals, axis=1, keepdims=True)
        idx = jnp.min(jnp.where(vals == m, iota_t, t), axis=1, keepdims=True)
        acc = acc + jnp.where(iota_ka == k, row_base + idx, 0)
        return jnp.where(iota_t == idx, neg_inf, vals), acc

    _, gidxa = lax.fori_loop(0, k_act, act_step,
                             (mag, jnp.zeros((b, ka_pad), jnp.int32)))
    gidxa_ref[...] = gidxa

    rev = jnp.max(mag, axis=1, keepdims=True) - mag

    def bkg_step(k, carry):
        vals, acc, msk = carry
        m = jnp.max(vals, axis=1, keepdims=True)
        idx = jnp.min(jnp.where(vals == m, iota_t, t), axis=1, keepdims=True)
        acc = acc + jnp.where(iota_kb == k, row_base + idx, 0)
        hit = (iota_t == idx)
        return jnp.where(hit, neg_inf, vals), acc, msk + hit.astype(_F32)

    _, gidxb, bkg_mask = lax.fori_loop(
        0, k_bkg, bkg_step,
        (rev, jnp.zeros((b, kb_pad), jnp.int32), jnp.zeros((b, t), _F32)))
    gidxb_ref[...] = gidxb

    sb = lax.dot_general(bkg_mask, casbt_ref[...], (((1,), (1,)), ((0,), (0,))),
                         precision=_HIGHEST,
                         preferred_element_type=_F32) / _F32(k_bkg)

    iota_ct = lax.broadcasted_iota(jnp.int32, (bc, t), 1)
    vals_scr[...] = casct_ref[...]

    def cas_step(k, acc):
        v = vals_scr[...]
        m = jnp.max(v, axis=1, keepdims=True)
        idx = jnp.min(jnp.where(v == m, iota_ct, t), axis=1, keepdims=True)
        vals_scr[...] = jnp.where(iota_ct == idx, neg_inf, v)
        return acc + m

    acc = lax.fori_loop(0, k_act, cas_step, jnp.zeros((bc, 1), _F32))
    sa = acc.reshape(sa_ref.shape) / _F32(k_act)

    def _softmax(v):
        e = jnp.exp(v - jnp.max(v, axis=1, keepdims=True))
        return e / jnp.sum(e, axis=1, keepdims=True)

    sa_ref[...] = _softmax(sa)
    sb_ref[...] = _softmax(sb)


def _stage_b(mag, cas_bt, cas_ct, k_act, k_bkg, ka_pad, kb_pad, *, interpret=False):
    b, t = mag.shape
    c = cas_bt.shape[2]
    bc = cas_ct.shape[0]
    return pl.pallas_call(
        functools.partial(_select_body, k_act, k_bkg, t),
        grid=(1,),
        in_specs=[
            pl.BlockSpec((b, t), lambda i: (0, 0)),
            pl.BlockSpec((b, t, c), lambda i: (0, 0, 0)),
            pl.BlockSpec((bc, t), lambda i: (0, 0)),
        ],
        out_specs=[
            pl.BlockSpec((b, ka_pad), lambda i: (0, 0)),
            pl.BlockSpec((b, kb_pad), lambda i: (0, 0)),
            pl.BlockSpec((b, c), lambda i: (0, 0)),
            pl.BlockSpec((b, c), lambda i: (0, 0)),
        ],
        out_shape=[
            jax.ShapeDtypeStruct((b, ka_pad), jnp.int32),
            jax.ShapeDtypeStruct((b, kb_pad), jnp.int32),
            jax.ShapeDtypeStruct((b, c), _F32),
            jax.ShapeDtypeStruct((b, c), _F32),
        ],
        scratch_shapes=[pltpu.VMEM((bc, t), _F32)],
        interpret=interpret,
    )(mag, cas_bt, cas_ct)


_SC_WORKERS = 32


def _sc_gather(table, idx_flat, chunk):
    n_rows, hid = idx_flat.shape[0], table.shape[1]
    rows_per = n_rows // _SC_WORKERS
    nchunks = rows_per // chunk
    mesh = plsc.VectorSubcoreMesh(core_axis_name="c", subcore_axis_name="s")

    @functools.partial(
        pl.kernel,
        out_type=jax.ShapeDtypeStruct((n_rows, hid), _F32),
        mesh=mesh,
        scratch_types=[
            pltpu.VMEM((chunk,), jnp.int32),
            pltpu.VMEM((chunk, hid), _F32),
            pltpu.SemaphoreType.DMA,
        ],
    )
    def gk(table_hbm, idx_hbm, out_hbm, idx_v, rows_v, sem):
        wid = lax.axis_index("s") * 2 + lax.axis_index("c")
        for h in range(nchunks):
            base = wid * rows_per + h * chunk
            pltpu.sync_copy(idx_hbm.at[pl.ds(base, chunk)], idx_v)
            pltpu.async_copy(table_hbm.at[idx_v], rows_v, sem).wait()
            pltpu.sync_copy(rows_v, out_hbm.at[pl.ds(base, chunk)])

    return gk(table, idx_flat)


def kernel(x, W1, b1, W2):
    b, t, d = x.shape
    hid = W1.shape[0]
    c = W2.shape[0]
    k_act = t // 8
    k_bkg = t // 6
    ka_pad = 96
    kb_pad = 128

    w_stack = jnp.transpose(W1, (2, 1, 0)).reshape(3 * d, hid)
    w_cas = jnp.transpose(W2[:, :, 0], (1, 0))
    b1_row = b1.reshape(1, hid)
    x_prev = jnp.pad(x, ((0, 0), (1, 0), (0, 0)))[:, :t].reshape(b * t, d)
    x_next = jnp.pad(x, ((0, 0), (0, 1), (0, 0)))[:, 1:].reshape(b * t, d)

    feats = _stage_a(x_prev, x.reshape(b * t, d), x_next, w_stack, b1_row)
    cas, cas_sm = _stage_a2(feats, w_cas)

    features = feats.reshape(b, t, hid)
    mag = jnp.linalg.norm(features, axis=2)

    cas_bt = cas.reshape(b, t, c)
    cas_ct = jnp.transpose(cas_bt, (0, 2, 1)).reshape(b * c, t)
    gidxa, gidxb, score_act, score_bkg = _stage_b(
        mag, cas_bt, cas_ct, k_act, k_bkg, ka_pad, kb_pad)

    act_rows = _sc_gather(feats, gidxa.reshape(b * ka_pad), 48)
    bkg_rows = _sc_gather(feats, gidxb.reshape(b * kb_pad), 32)

    feat_act = act_rows.reshape(b, ka_pad, hid)[:, :k_act]
    feat_bkg = bkg_rows.reshape(b, kb_pad, hid)[:, :k_bkg]
    cas_softmax = cas_sm.reshape(b, t, c)
    return (score_act, score_bkg, feat_act, feat_bkg, features, cas_softmax)


# speedup vs baseline: 1.2352x; 1.0182x over previous
import functools

import jax
import jax.numpy as jnp
from jax import lax
from jax.experimental import pallas as pl
from jax.experimental.pallas import tpu as pltpu
from jax.experimental.pallas import tpu_sc as plsc

_F32 = jnp.float32
_HIGHEST = lax.Precision.HIGHEST


def _dot(a, b):
    return lax.dot_general(a, b, (((1,), (0,)), ((), ())),
                           precision=lax.Precision.DEFAULT,
                           preferred_element_type=_F32)


def _conv_block_body(xp_ref, xc_ref, xn_ref, w_ref, b_ref, feat_ref, xs_scr):
    d = xp_ref.shape[1]
    xs_scr[:, 0:d] = xp_ref[...]
    xs_scr[:, d:2 * d] = xc_ref[...]
    xs_scr[:, 2 * d:3 * d] = xn_ref[...]
    acc = _dot(xs_scr[...], w_ref[...])
    feat_ref[...] = jnp.maximum(acc + b_ref[...], 0.0)


def _stage_a(x_prev, x_cur, x_next, w_stack, b1_row, *, interpret=False):
    m_rows, d = x_cur.shape
    hid = w_stack.shape[1]
    bt = 96
    x_spec = pl.BlockSpec((bt, d), lambda i: (i, 0))
    return pl.pallas_call(
        _conv_block_body,
        grid=(m_rows // bt,),
        in_specs=[
            x_spec, x_spec, x_spec,
            pl.BlockSpec((3 * d, hid), lambda i: (0, 0)),
            pl.BlockSpec((1, hid), lambda i: (0, 0)),
        ],
        out_specs=pl.BlockSpec((bt, hid), lambda i: (i, 0)),
        out_shape=jax.ShapeDtypeStruct((m_rows, hid), _F32),
        scratch_shapes=[pltpu.VMEM((bt, 3 * d), _F32)],
        interpret=interpret,
    )(x_prev, x_cur, x_next, w_stack, b1_row)


def _cas_block_body(f_ref, wc_ref, cas_ref, sm_ref):
    c = _dot(f_ref[...], wc_ref[...])
    cas_ref[...] = c
    m = jnp.max(c, axis=1, keepdims=True)
    e = jnp.exp(c - m)
    sm_ref[...] = e / jnp.sum(e, axis=1, keepdims=True)


def _stage_a2(feats, w_cas, *, interpret=False):
    m_rows, hid = feats.shape
    c = w_cas.shape[1]
    bt = 600
    return pl.pallas_call(
        _cas_block_body,
        grid=(m_rows // bt,),
        in_specs=[
            pl.BlockSpec((bt, hid), lambda i: (i, 0)),
            pl.BlockSpec((hid, c), lambda i: (0, 0)),
        ],
        out_specs=[
            pl.BlockSpec((bt, c), lambda i: (i, 0)),
            pl.BlockSpec((bt, c), lambda i: (i, 0)),
        ],
        out_shape=[
            jax.ShapeDtypeStruct((m_rows, c), _F32),
            jax.ShapeDtypeStruct((m_rows, c), _F32),
        ],
        interpret=interpret,
    )(feats, w_cas)


def _select_body(k_act, k_bkg, t, mag_ref, casbt_ref, casct_ref,
                 gidxa_ref, gidxb_ref, sa_ref, sb_ref, vals_scr):
    b = mag_ref.shape[0]
    bc = casct_ref.shape[0]
    ka_pad = gidxa_ref.shape[1]
    kb_pad = gidxb_ref.shape[1]
    neg_inf = _F32(-jnp.inf)
    iota_t = lax.broadcasted_iota(jnp.int32, (b, t), 1)
    iota_ka = lax.broadcasted_iota(jnp.int32, (b, ka_pad), 1)
    iota_kb = lax.broadcasted_iota(jnp.int32, (b, kb_pad), 1)
    row_base = lax.broadcasted_iota(jnp.int32, (b, 1), 0) * t

    mag = mag_ref[...]

    def act_step(k, carry):
        vals, acc = carry
        m = jnp.max(vals, axis=1, keepdims=True)
        idx = jnp.min(jnp.where(vals == m, iota_t, t), axis=1, keepdims=True)
        acc = acc + jnp.where(iota_ka == k, row_base + idx, 0)
        return jnp.where(iota_t == idx, neg_inf, vals), acc

    _, gidxa = lax.fori_loop(0, k_act, act_step,
                             (mag, jnp.zeros((b, ka_pad), jnp.int32)))
    gidxa_ref[...] = gidxa

    rev = jnp.max(mag, axis=1, keepdims=True) - mag

    def bkg_step(k, carry):
        vals, acc, msk = carry
        m = jnp.max(vals, axis=1, keepdims=True)
        idx = jnp.min(jnp.where(vals == m, iota_t, t), axis=1, keepdims=True)
        acc = acc + jnp.where(iota_kb == k, row_base + idx, 0)
        hit = (iota_t == idx)
        return jnp.where(hit, neg_inf, vals), acc, msk + hit.astype(_F32)

    _, gidxb, bkg_mask = lax.fori_loop(
        0, k_bkg, bkg_step,
        (rev, jnp.zeros((b, kb_pad), jnp.int32), jnp.zeros((b, t), _F32)))
    gidxb_ref[...] = gidxb

    sb = lax.dot_general(bkg_mask, casbt_ref[...], (((1,), (1,)), ((0,), (0,))),
                         precision=_HIGHEST,
                         preferred_element_type=_F32) / _F32(k_bkg)

    ct = casct_ref[...]
    vals_scr[...] = ct
    lo0 = jnp.min(ct, axis=1, keepdims=True)
    hi0 = jnp.max(ct, axis=1, keepdims=True)

    def bis_step(_, carry):
        lo, hi = carry
        mid = 0.5 * (lo + hi)
        cnt = jnp.sum((vals_scr[...] > mid).astype(_F32), axis=1,
                      keepdims=True)
        take_hi = cnt >= _F32(k_act)
        return jnp.where(take_hi, mid, lo), jnp.where(take_hi, hi, mid)

    lo, _ = lax.fori_loop(0, 30, bis_step, (lo0, hi0))
    v = vals_scr[...]
    above = (v > lo).astype(_F32)
    cnt = jnp.sum(above, axis=1, keepdims=True)
    ssum = jnp.sum(v * above, axis=1, keepdims=True)
    ssum = ssum + (_F32(k_act) - cnt) * lo
    sa = ssum.reshape(sa_ref.shape) / _F32(k_act)

    def _softmax(v):
        e = jnp.exp(v - jnp.max(v, axis=1, keepdims=True))
        return e / jnp.sum(e, axis=1, keepdims=True)

    sa_ref[...] = _softmax(sa)
    sb_ref[...] = _softmax(sb)


def _stage_b(mag, cas_bt, cas_ct, k_act, k_bkg, ka_pad, kb_pad, *, interpret=False):
    b, t = mag.shape
    c = cas_bt.shape[2]
    bc = cas_ct.shape[0]
    return pl.pallas_call(
        functools.partial(_select_body, k_act, k_bkg, t),
        grid=(1,),
        in_specs=[
            pl.BlockSpec((b, t), lambda i: (0, 0)),
            pl.BlockSpec((b, t, c), lambda i: (0, 0, 0)),
            pl.BlockSpec((bc, t), lambda i: (0, 0)),
        ],
        out_specs=[
            pl.BlockSpec((b, ka_pad), lambda i: (0, 0)),
            pl.BlockSpec((b, kb_pad), lambda i: (0, 0)),
            pl.BlockSpec((b, c), lambda i: (0, 0)),
            pl.BlockSpec((b, c), lambda i: (0, 0)),
        ],
        out_shape=[
            jax.ShapeDtypeStruct((b, ka_pad), jnp.int32),
            jax.ShapeDtypeStruct((b, kb_pad), jnp.int32),
            jax.ShapeDtypeStruct((b, c), _F32),
            jax.ShapeDtypeStruct((b, c), _F32),
        ],
        scratch_shapes=[pltpu.VMEM((bc, t), _F32)],
        interpret=interpret,
    )(mag, cas_bt, cas_ct)


_SC_WORKERS = 32


def _sc_gather(table, idx_flat, chunk):
    n_rows, hid = idx_flat.shape[0], table.shape[1]
    rows_per = n_rows // _SC_WORKERS
    nchunks = rows_per // chunk
    mesh = plsc.VectorSubcoreMesh(core_axis_name="c", subcore_axis_name="s")

    @functools.partial(
        pl.kernel,
        out_type=jax.ShapeDtypeStruct((n_rows, hid), _F32),
        mesh=mesh,
        scratch_types=[
            pltpu.VMEM((chunk,), jnp.int32),
            pltpu.VMEM((chunk, hid), _F32),
            pltpu.SemaphoreType.DMA,
        ],
    )
    def gk(table_hbm, idx_hbm, out_hbm, idx_v, rows_v, sem):
        wid = lax.axis_index("s") * 2 + lax.axis_index("c")
        for h in range(nchunks):
            base = wid * rows_per + h * chunk
            pltpu.sync_copy(idx_hbm.at[pl.ds(base, chunk)], idx_v)
            pltpu.async_copy(table_hbm.at[idx_v], rows_v, sem).wait()
            pltpu.sync_copy(rows_v, out_hbm.at[pl.ds(base, chunk)])

    return gk(table, idx_flat)


def kernel(x, W1, b1, W2):
    b, t, d = x.shape
    hid = W1.shape[0]
    c = W2.shape[0]
    k_act = t // 8
    k_bkg = t // 6
    ka_pad = 96
    kb_pad = 128

    w_stack = jnp.transpose(W1, (2, 1, 0)).reshape(3 * d, hid)
    w_cas = jnp.transpose(W2[:, :, 0], (1, 0))
    b1_row = b1.reshape(1, hid)
    x_prev = jnp.pad(x, ((0, 0), (1, 0), (0, 0)))[:, :t].reshape(b * t, d)
    x_next = jnp.pad(x, ((0, 0), (0, 1), (0, 0)))[:, 1:].reshape(b * t, d)

    feats = _stage_a(x_prev, x.reshape(b * t, d), x_next, w_stack, b1_row)
    cas, cas_sm = _stage_a2(feats, w_cas)

    features = feats.reshape(b, t, hid)
    mag = jnp.linalg.norm(features, axis=2)

    cas_bt = cas.reshape(b, t, c)
    cas_ct = jnp.transpose(cas_bt, (0, 2, 1)).reshape(b * c, t)
    gidxa, gidxb, score_act, score_bkg = _stage_b(
        mag, cas_bt, cas_ct, k_act, k_bkg, ka_pad, kb_pad)

    act_rows = _sc_gather(feats, gidxa.reshape(b * ka_pad), 48)
    bkg_rows = _sc_gather(feats, gidxb.reshape(b * kb_pad), 32)

    feat_act = act_rows.reshape(b, ka_pad, hid)[:, :k_act]
    feat_bkg = bkg_rows.reshape(b, kb_pad, hid)[:, :k_bkg]
    cas_softmax = cas_sm.reshape(b, t, c)
    return (score_act, score_bkg, feat_act, feat_bkg, features, cas_softmax)
